# baseline (speedup 1.0000x reference)
import jax
import jax.numpy as jnp
from jax.experimental import pallas as pl
from jax.experimental.pallas import tpu as pltpu
from jax.experimental.pallas import tpu_sc as plsc

B, N, D_IN, CODE_DIM, K = 64, 576, 384, 64, 8192
M = B * N
TM = 1024
KW = 4096
TM_DEC = 1024
GW = 128
PAD_W = 128

_DN = (((1,), (1,)), ((), ()))


def _tree_rowsum(t):
    w = t.shape[1]
    while w > 1:
        t = t[:, : w // 2] + t[:, w // 2 : w]
        w //= 2
    return t


def _enc_argmin_body(x_ref, we_ref, cb_ref, csq_ref, iota_ref, enc_ref, code_ref):
    enc = jnp.dot(x_ref[...], we_ref[...], preferred_element_type=jnp.float32)
    enc_ref[...] = enc
    lhs = (2.0 * enc).astype(jnp.bfloat16)
    a = _tree_rowsum(enc * enc)
    acc = jnp.full((TM, 1), jnp.inf, jnp.bfloat16)
    idx = jnp.zeros((TM, 1), jnp.int32)
    for t in range(K // KW):
        conv = jax.lax.dot_general(lhs, cb_ref[t * KW:(t + 1) * KW, :], _DN,
                                   preferred_element_type=jnp.float32)
        dist = (a - conv) + csq_ref[:, t * KW:(t + 1) * KW]
        m = jnp.min(dist, axis=1, keepdims=True)
        iota = iota_ref[:, :KW]
        icf = jnp.min(jnp.where(dist <= m, iota, float(KW)), axis=1,
                      keepdims=True)
        ic = icf.astype(jnp.int32) + t * KW
        accv = acc.astype(jnp.float32)
        take = m < accv
        acc = jnp.where(take, m, accv).astype(jnp.bfloat16)
        idx = jnp.where(take, ic, idx)
    code_ref[...] = idx.reshape(1, TM)


def _dec_body(enc_ref, nrp_ref, wd_ref, out_ref, nr_ref):
    nr = nrp_ref[:, :CODE_DIM]
    nr_ref[...] = nr
    st = enc_ref[...] + (nr - enc_ref[...])
    out_ref[...] = jnp.dot(st.astype(jnp.bfloat16), wd_ref[...],
                           preferred_element_type=jnp.float32)


def _sc_gather(cb_padded, code_flat):
    idx = code_flat.reshape(1, M)
    mesh = plsc.VectorSubcoreMesh(core_axis_name="core", subcore_axis_name="subcore")

    @pl.kernel(out_type=jax.ShapeDtypeStruct((M, PAD_W), cb_padded.dtype),
               mesh=mesh)
    def gather_kernel(cb_hbm, i_hbm, o_hbm):
        def body(i_vmem, o_vmem):
            pltpu.sync_copy(cb_hbm.at[i_vmem.at[0]], o_vmem)

        pltpu.emit_pipeline(
            body,
            grid=(M // GW,),
            in_specs=[pl.BlockSpec((1, GW), index_map=lambda i: (0, i))],
            out_specs=[pl.BlockSpec((GW, PAD_W), index_map=lambda i: (i, 0))],
            core_axis_name=("core", "subcore"),
            dimension_semantics=(pltpu.PARALLEL,),
        )(i_hbm, o_hbm)

    return gather_kernel(cb_padded, idx)


def kernel(x, W_enc, codebook, W_dec):
    xf_bf = x.reshape(M, D_IN).astype(jnp.bfloat16)
    we_bf = W_enc.astype(jnp.bfloat16)
    cb_bf = codebook.astype(jnp.bfloat16)
    csq = jnp.sum(codebook * codebook, axis=1)[None, :]
    iota_row = jnp.arange(KW, dtype=jnp.float32)[None, :]

    enc_flat, code_2d = pl.pallas_call(
        _enc_argmin_body,
        grid=(M // TM,),
        in_specs=[
            pl.BlockSpec((TM, D_IN), lambda i: (i, 0)),
            pl.BlockSpec((D_IN, CODE_DIM), lambda i: (0, 0)),
            pl.BlockSpec((K, CODE_DIM), lambda i: (0, 0)),
            pl.BlockSpec((1, K), lambda i: (0, 0)),
            pl.BlockSpec((1, KW), lambda i: (0, 0)),
        ],
        out_specs=[
            pl.BlockSpec((TM, CODE_DIM), lambda i: (i, 0)),
            pl.BlockSpec((1, TM), lambda i: (0, i)),
        ],
        out_shape=[
            jax.ShapeDtypeStruct((M, CODE_DIM), jnp.float32),
            jax.ShapeDtypeStruct((1, M), jnp.int32),
        ],
        compiler_params=pltpu.CompilerParams(
            dimension_semantics=("arbitrary",)),
    )(xf_bf, we_bf, cb_bf, csq, iota_row)

    code_flat = code_2d.reshape(M)
    cb_padded = jnp.pad(codebook, ((0, 0), (0, PAD_W - CODE_DIM)))
    new_rep_padded = _sc_gather(cb_padded, code_flat)

    dec_flat, new_rep_flat = pl.pallas_call(
        _dec_body,
        grid=(M // TM_DEC,),
        in_specs=[
            pl.BlockSpec((TM_DEC, CODE_DIM), lambda i: (i, 0)),
            pl.BlockSpec((TM_DEC, PAD_W), lambda i: (i, 0)),
            pl.BlockSpec((CODE_DIM, D_IN), lambda i: (0, 0)),
        ],
        out_specs=[
            pl.BlockSpec((TM_DEC, D_IN), lambda i: (i, 0)),
            pl.BlockSpec((TM_DEC, CODE_DIM), lambda i: (i, 0)),
        ],
        out_shape=[
            jax.ShapeDtypeStruct((M, D_IN), jnp.float32),
            jax.ShapeDtypeStruct((M, CODE_DIM), jnp.float32),
        ],
        compiler_params=pltpu.CompilerParams(
            dimension_semantics=("arbitrary",)),
    )(enc_flat, new_rep_padded, W_dec.astype(jnp.bfloat16))

    enc_oup = enc_flat.reshape(B, N, CODE_DIM)
    code = code_flat.reshape(B, N)
    new_representation = new_rep_flat.reshape(B, N, CODE_DIM)
    dec_oup = dec_flat.reshape(B, N, D_IN)
    return enc_oup, code, new_representation, dec_oup


# speedup vs baseline: 1.0241x; 1.0241x over previous
import jax
import jax.numpy as jnp
from jax.experimental import pallas as pl
from jax.experimental.pallas import tpu as pltpu
from jax.experimental.pallas import tpu_sc as plsc

B, N, D_IN, CODE_DIM, K = 64, 576, 384, 64, 8192
M = B * N
TM = 1536
KW = 4096
TM_DEC = 1024
GW = 128
PAD_W = 128

_DN = (((1,), (1,)), ((), ()))


def _tree_rowsum(t):
    w = t.shape[1]
    while w > 1:
        t = t[:, : w // 2] + t[:, w // 2 : w]
        w //= 2
    return t


def _enc_argmin_body(x_ref, we_ref, cb_ref, csq_ref, iota_ref, enc_ref, code_ref):
    enc = jnp.dot(x_ref[...], we_ref[...], preferred_element_type=jnp.float32)
    enc_ref[...] = enc
    lhs = (2.0 * enc).astype(jnp.bfloat16)
    a = _tree_rowsum(enc * enc)
    acc = jnp.full((TM, 1), jnp.inf, jnp.bfloat16)
    idx = jnp.zeros((TM, 1), jnp.int32)
    for t in range(K // KW):
        conv = jax.lax.dot_general(lhs, cb_ref[t * KW:(t + 1) * KW, :], _DN,
                                   preferred_element_type=jnp.float32)
        dist = (a - conv) + csq_ref[:, t * KW:(t + 1) * KW]
        m = jnp.min(dist, axis=1, keepdims=True)
        iota = iota_ref[:, :KW]
        icf = jnp.min(jnp.where(dist <= m, iota, float(KW)), axis=1,
                      keepdims=True)
        ic = icf.astype(jnp.int32) + t * KW
        accv = acc.astype(jnp.float32)
        take = m < accv
        acc = jnp.where(take, m, accv).astype(jnp.bfloat16)
        idx = jnp.where(take, ic, idx)
    code_ref[...] = idx.reshape(1, TM)


def _dec_body(enc_ref, nr_ref, wd_ref, out_ref):
    st = enc_ref[...] + (nr_ref[...] - enc_ref[...])
    out_ref[...] = jnp.dot(st.astype(jnp.bfloat16), wd_ref[...],
                           preferred_element_type=jnp.float32)


def _sc_gather(cb_padded, code_flat):
    idx = code_flat.reshape(1, M)
    mesh = plsc.VectorSubcoreMesh(core_axis_name="core", subcore_axis_name="subcore")

    @pl.kernel(out_type=jax.ShapeDtypeStruct((M, PAD_W), cb_padded.dtype),
               mesh=mesh)
    def gather_kernel(cb_hbm, i_hbm, o_hbm):
        def body(i_vmem, o_vmem):
            pltpu.sync_copy(cb_hbm.at[i_vmem.at[0]], o_vmem)

        pltpu.emit_pipeline(
            body,
            grid=(M // GW,),
            in_specs=[pl.BlockSpec((1, GW), index_map=lambda i: (0, i))],
            out_specs=[pl.BlockSpec((GW, PAD_W), index_map=lambda i: (i, 0))],
            core_axis_name=("core", "subcore"),
            dimension_semantics=(pltpu.PARALLEL,),
        )(i_hbm, o_hbm)

    return gather_kernel(cb_padded, idx)


def kernel(x, W_enc, codebook, W_dec):
    xf_bf = x.reshape(M, D_IN).astype(jnp.bfloat16)
    we_bf = W_enc.astype(jnp.bfloat16)
    cb_bf = codebook.astype(jnp.bfloat16)
    csq = jnp.sum(codebook * codebook, axis=1)[None, :]
    iota_row = jnp.arange(KW, dtype=jnp.float32)[None, :]

    enc_flat, code_2d = pl.pallas_call(
        _enc_argmin_body,
        grid=(M // TM,),
        in_specs=[
            pl.BlockSpec((TM, D_IN), lambda i: (i, 0)),
            pl.BlockSpec((D_IN, CODE_DIM), lambda i: (0, 0)),
            pl.BlockSpec((K, CODE_DIM), lambda i: (0, 0)),
            pl.BlockSpec((1, K), lambda i: (0, 0)),
            pl.BlockSpec((1, KW), lambda i: (0, 0)),
        ],
        out_specs=[
            pl.BlockSpec((TM, CODE_DIM), lambda i: (i, 0)),
            pl.BlockSpec((1, TM), lambda i: (0, i)),
        ],
        out_shape=[
            jax.ShapeDtypeStruct((M, CODE_DIM), jnp.float32),
            jax.ShapeDtypeStruct((1, M), jnp.int32),
        ],
        compiler_params=pltpu.CompilerParams(
            dimension_semantics=("arbitrary",)),
    )(xf_bf, we_bf, cb_bf, csq, iota_row)

    code_flat = code_2d.reshape(M)
    cb_padded = jnp.pad(codebook, ((0, 0), (0, PAD_W - CODE_DIM)))
    new_rep_flat = _sc_gather(cb_padded, code_flat)[:, :CODE_DIM]

    dec_flat = pl.pallas_call(
        _dec_body,
        grid=(M // TM_DEC,),
        in_specs=[
            pl.BlockSpec((TM_DEC, CODE_DIM), lambda i: (i, 0)),
            pl.BlockSpec((TM_DEC, CODE_DIM), lambda i: (i, 0)),
            pl.BlockSpec((CODE_DIM, D_IN), lambda i: (0, 0)),
        ],
        out_specs=pl.BlockSpec((TM_DEC, D_IN), lambda i: (i, 0)),
        out_shape=jax.ShapeDtypeStruct((M, D_IN), jnp.float32),
        compiler_params=pltpu.CompilerParams(
            dimension_semantics=("arbitrary",)),
    )(enc_flat, new_rep_flat, W_dec.astype(jnp.bfloat16))

    enc_oup = enc_flat.reshape(B, N, CODE_DIM)
    code = code_flat.reshape(B, N)
    new_representation = new_rep_flat.reshape(B, N, CODE_DIM)
    dec_oup = dec_flat.reshape(B, N, D_IN)
    return enc_oup, code, new_representation, dec_oup
